# Initial kernel scaffold; baseline (speedup 1.0000x reference)
import jax
import jax.numpy as jnp
from jax.experimental import pallas as pl


def kernel(x, edge_index, W, b, gamma, beta):
    raise NotImplementedError("write your pallas kernel here")


# speedup vs baseline: 5.8855x; 5.8855x over previous
import functools

import jax
import jax.numpy as jnp
from jax import lax
from jax.experimental import pallas as pl
from jax.experimental.pallas import tpu as pltpu
from jax.experimental.pallas import tpu_sc as plsc

N = 10000
E = 320000
D = 128
DH = D // 2
EPS = 1e-5

NC = 2
NS = 16

_sc_mesh = plsc.VectorSubcoreMesh(core_axis_name="c", subcore_axis_name="s")

CHD = 80
EPT_DEG = E // NS
NCH_DEG = EPT_DEG // CHD


@functools.partial(
    pl.kernel,
    out_type=jax.ShapeDtypeStruct((NC * N,), jnp.float32),
    mesh=_sc_mesh,
    scratch_types=[
        pltpu.VMEM((NCH_DEG, CHD), jnp.int32),
        pltpu.VMEM((CHD,), jnp.float32),
        pltpu.VMEM((1000,), jnp.float32),
        pltpu.VMEM((N,), jnp.float32),
        pltpu.VMEM_SHARED((N,), jnp.float32),
    ],
)
def _deg_kernel(edge_blk, z1, deg_out, idx_v, ones_v, zv, dv, deg_s):
    c = lax.axis_index("c")
    s = lax.axis_index("s")
    for i in range(CHD // 16):
        ones_v[pl.ds(i * 16, 16)] = jnp.ones((16,), jnp.float32)
    pltpu.sync_copy(edge_blk.at[c * NS + s], idx_v)
    @pl.when(s < 10)
    def _():
        pltpu.sync_copy(z1.at[pl.ds(s * 1000, 1000)], zv)
        pltpu.sync_copy(zv, deg_s.at[pl.ds(s * 1000, 1000)])
    plsc.subcore_barrier()

    @pl.loop(0, NCH_DEG)
    def _(j):
        pltpu.sync_copy(ones_v, deg_s.at[idx_v.at[j]], add=True)

    plsc.subcore_barrier()

    @pl.when(s == 0)
    def _():
        pltpu.sync_copy(deg_s, dv)
        pltpu.sync_copy(dv, deg_out.at[pl.ds(c * N, N)])


CHA = 125
EPT = E // NS
NCHT = EPT // CHA
ZT = 10
RPZ = N // ZT
SRW = 200


@functools.partial(
    pl.kernel,
    out_type=jax.ShapeDtypeStruct((NC * N, DH), jnp.float32),
    mesh=_sc_mesh,
    scratch_types=[
        pltpu.VMEM((NCHT, CHA), jnp.int32),
        pltpu.VMEM((NCHT, CHA), jnp.int32),
        pltpu.VMEM((CHA, DH), jnp.float32),
        pltpu.VMEM((SRW, DH), jnp.float32),
        pltpu.VMEM_SHARED((N, DH), jnp.float32),
        pltpu.SemaphoreType.DMA,
    ],
    compiler_params=pltpu.CompilerParams(use_tc_tiling_on_sc=False),
)
def _agg_kernel(feat_lo, feat_hi, src2, dst2, zh, part,
                sidx_v, didx_v, rows_v, stage_v, agg_s, gsem):
    c = lax.axis_index("c")
    s = lax.axis_index("s")
    pltpu.sync_copy(src2.at[pl.ds(s * NCHT, NCHT)], sidx_v)
    pltpu.sync_copy(dst2.at[pl.ds(s * NCHT, NCHT)], didx_v)
    @pl.when(s < ZT)
    def _():
        for r in range(RPZ // SRW):
            off = pl.ds(s * RPZ + r * SRW, SRW)
            pltpu.sync_copy(zh.at[off], stage_v)
            pltpu.sync_copy(stage_v, agg_s.at[off])
    plsc.subcore_barrier()

    def edge_pass(ftab):
        @pl.loop(0, NCHT)
        def _(j):
            pltpu.async_copy(ftab.at[sidx_v.at[j]], rows_v, gsem).wait()
            pltpu.sync_copy(rows_v, agg_s.at[didx_v.at[j]], add=True)

    @pl.when(c == 0)
    def _():
        edge_pass(feat_lo)

    @pl.when(c == 1)
    def _():
        edge_pass(feat_hi)

    plsc.subcore_barrier()

    @pl.when(s < ZT)
    def _():
        for r in range(RPZ // SRW):
            pltpu.sync_copy(agg_s.at[pl.ds(s * RPZ + r * SRW, SRW)], stage_v)
            pltpu.sync_copy(stage_v,
                            part.at[pl.ds(c * N + s * RPZ + r * SRW, SRW)])


def _feat_body(x_ref, dsrc_ref, ddst_ref, flo_ref, fhi_ref, ndst_ref):
    nsrc = lax.rsqrt(jnp.maximum(dsrc_ref[...], 1.0))
    flo_ref[...] = x_ref[:, :DH] * nsrc
    fhi_ref[...] = x_ref[:, DH:] * nsrc
    ndst_ref[...] = lax.rsqrt(jnp.maximum(ddst_ref[...], 1.0))


_feat_call = pl.pallas_call(
    _feat_body,
    out_shape=[
        jax.ShapeDtypeStruct((N, DH), jnp.float32),
        jax.ShapeDtypeStruct((N, DH), jnp.float32),
        jax.ShapeDtypeStruct((N, 1), jnp.float32),
    ],
)


def _final_body(p_ref, w_ref, b_ref, x_ref, ndst_ref, g_ref, be_ref, o_ref):
    rst = (jnp.dot(p_ref[0], w_ref[0], preferred_element_type=jnp.float32)
           + jnp.dot(p_ref[1], w_ref[1], preferred_element_type=jnp.float32))
    rst = rst * ndst_ref[...] + b_ref[...]
    mean = jnp.mean(rst, axis=0, keepdims=True)
    var = jnp.mean(rst * rst, axis=0, keepdims=True) - mean * mean
    o_ref[...] = ((rst - mean) * lax.rsqrt(var + EPS) * g_ref[...]
                  + be_ref[...] + x_ref[...])


_final_call = pl.pallas_call(
    _final_body,
    out_shape=jax.ShapeDtypeStruct((N, D), jnp.float32),
)


def kernel(x, edge_index, W, b, gamma, beta):
    edge_blk = edge_index.reshape(2 * NS, NCH_DEG, CHD)
    src2 = edge_index[0].reshape(E // CHA, CHA)
    dst2 = edge_index[1].reshape(E // CHA, CHA)
    z1 = jnp.zeros((N,), jnp.float32)
    zh = jnp.zeros((N, DH), jnp.float32)

    deg = _deg_kernel(edge_blk, z1)
    dsrc = deg[:N].reshape(N, 1)
    ddst = deg[N:].reshape(N, 1)
    flo, fhi, ndst = _feat_call(x, dsrc, ddst)
    part = _agg_kernel(flo, fhi, src2, dst2, zh).reshape(NC, N, DH)
    w2 = W.reshape(2, DH, D)
    return _final_call(part, w2, b.reshape(1, D), x, ndst,
                       gamma.reshape(1, D), beta.reshape(1, D))


# speedup vs baseline: 9.3133x; 1.5824x over previous
import functools

import jax
import jax.numpy as jnp
from jax import lax
from jax.experimental import pallas as pl
from jax.experimental.pallas import tpu as pltpu
from jax.experimental.pallas import tpu_sc as plsc

N = 10000
E = 320000
D = 128
DH = D // 2
EPS = 1e-5

NC = 2
NS = 16

_sc_mesh = plsc.VectorSubcoreMesh(core_axis_name="c", subcore_axis_name="s")

CHA = 125
EPT = E // NS
NCHT = EPT // CHA
DEG_G = 8


@functools.partial(
    pl.kernel,
    out_type=jax.ShapeDtypeStruct((NC * N,), jnp.float32),
    mesh=_sc_mesh,
    scratch_types=[
        pltpu.VMEM((NCHT, CHA), jnp.int32),
        pltpu.VMEM((128,), jnp.float32),
        pltpu.VMEM((1000,), jnp.float32),
        pltpu.VMEM((N,), jnp.float32),
        pltpu.VMEM_SHARED((N,), jnp.float32),
        pltpu.SemaphoreType.DMA,
    ],
    compiler_params=pltpu.CompilerParams(use_tc_tiling_on_sc=False),
)
def _deg_kernel(src2, dst2, z1, deg_out, idx_v, ones_v, zv, dv, deg_s, sem):
    c = lax.axis_index("c")
    s = lax.axis_index("s")
    for i in range(128 // 16):
        ones_v[pl.ds(i * 16, 16)] = jnp.ones((16,), jnp.float32)
    ones_r = ones_v.at[pl.ds(0, CHA)]
    @pl.when(c == 0)
    def _():
        pltpu.sync_copy(src2.at[pl.ds(s * NCHT, NCHT)], idx_v)

    @pl.when(c == 1)
    def _():
        pltpu.sync_copy(dst2.at[pl.ds(s * NCHT, NCHT)], idx_v)
    @pl.when(s < 10)
    def _():
        pltpu.sync_copy(z1.at[pl.ds(s * 1000, 1000)], zv)
        pltpu.sync_copy(zv, deg_s.at[pl.ds(s * 1000, 1000)])
    plsc.subcore_barrier()

    @pl.loop(0, NCHT, step=DEG_G)
    def _(jb):
        for g in range(DEG_G):
            pltpu.async_copy(ones_r, deg_s.at[idx_v.at[jb + g]], sem, add=True)
        for g in range(DEG_G):
            pltpu.make_async_copy(ones_r, deg_s.at[idx_v.at[jb + g]], sem).wait()

    plsc.subcore_barrier()

    @pl.when(s == 0)
    def _():
        pltpu.sync_copy(deg_s, dv)
        pltpu.sync_copy(dv, deg_out.at[pl.ds(c * N, N)])


ZT = 10
RPZ = N // ZT
SRW = 200
NBUF = 4
PF = 2


@functools.partial(
    pl.kernel,
    out_type=jax.ShapeDtypeStruct((NC * N, DH), jnp.float32),
    mesh=_sc_mesh,
    scratch_types=[
        pltpu.VMEM((NCHT, CHA), jnp.int32),
        pltpu.VMEM((NCHT, CHA), jnp.int32),
        pltpu.VMEM((NBUF, CHA, DH), jnp.float32),
        pltpu.VMEM((SRW, DH), jnp.float32),
        pltpu.VMEM_SHARED((N, DH), jnp.float32),
    ] + [pltpu.SemaphoreType.DMA] * (2 * NBUF),
    compiler_params=pltpu.CompilerParams(use_tc_tiling_on_sc=False),
)
def _agg_kernel(feat_lo, feat_hi, src2, dst2, zh, part,
                sidx_v, didx_v, rows_v, stage_v, agg_s, *sems):
    gsem = sems[:NBUF]
    ssem = sems[NBUF:]
    c = lax.axis_index("c")
    s = lax.axis_index("s")
    pltpu.sync_copy(src2.at[pl.ds(s * NCHT, NCHT)], sidx_v)
    pltpu.sync_copy(dst2.at[pl.ds(s * NCHT, NCHT)], didx_v)
    @pl.when(s < ZT)
    def _():
        for r in range(RPZ // SRW):
            off = pl.ds(s * RPZ + r * SRW, SRW)
            pltpu.sync_copy(zh.at[off], stage_v)
            pltpu.sync_copy(stage_v, agg_s.at[off])
    plsc.subcore_barrier()

    def edge_pass(ftab):
        def start_gather(j, b):
            pltpu.async_copy(ftab.at[sidx_v.at[j]], rows_v.at[b], gsem[b])

        def wait_gather(j, b):
            pltpu.make_async_copy(ftab.at[sidx_v.at[j]], rows_v.at[b],
                                  gsem[b]).wait()

        def start_scatter(j, b):
            pltpu.async_copy(rows_v.at[b], agg_s.at[didx_v.at[j]], ssem[b],
                             add=True)

        def wait_scatter(j, b):
            pltpu.make_async_copy(rows_v.at[b], agg_s.at[didx_v.at[j]],
                                  ssem[b]).wait()

        for b in range(PF):
            start_gather(b, b)

        @pl.loop(0, NCHT, step=NBUF)
        def _(jb):
            for bb in range(NBUF):
                j = jb + bb
                nb = (bb + PF) % NBUF
                @pl.when(j + PF < NCHT)
                def _(j=j, nb=nb):
                    @pl.when(j + PF - NBUF >= 0)
                    def _():
                        wait_scatter(j + PF - NBUF, nb)
                    start_gather(j + PF, nb)
                wait_gather(j, bb)
                start_scatter(j, bb)

        for bb in range(NBUF):
            j = NCHT - NBUF + bb
            wait_scatter(j, j % NBUF)

    @pl.when(c == 0)
    def _():
        edge_pass(feat_lo)

    @pl.when(c == 1)
    def _():
        edge_pass(feat_hi)

    plsc.subcore_barrier()

    @pl.when(s < ZT)
    def _():
        for r in range(RPZ // SRW):
            pltpu.sync_copy(agg_s.at[pl.ds(s * RPZ + r * SRW, SRW)], stage_v)
            pltpu.sync_copy(stage_v,
                            part.at[pl.ds(c * N + s * RPZ + r * SRW, SRW)])


def _feat_body(x_ref, dsrc_ref, ddst_ref, flo_ref, fhi_ref, ndst_ref):
    nsrc = lax.rsqrt(jnp.maximum(dsrc_ref[...], 1.0))
    flo_ref[...] = x_ref[:, :DH] * nsrc
    fhi_ref[...] = x_ref[:, DH:] * nsrc
    ndst_ref[...] = lax.rsqrt(jnp.maximum(ddst_ref[...], 1.0))


_feat_call = pl.pallas_call(
    _feat_body,
    out_shape=[
        jax.ShapeDtypeStruct((N, DH), jnp.float32),
        jax.ShapeDtypeStruct((N, DH), jnp.float32),
        jax.ShapeDtypeStruct((N, 1), jnp.float32),
    ],
)


def _final_body(p_ref, w_ref, b_ref, x_ref, ndst_ref, g_ref, be_ref, o_ref):
    rst = (jnp.dot(p_ref[0], w_ref[0], preferred_element_type=jnp.float32)
           + jnp.dot(p_ref[1], w_ref[1], preferred_element_type=jnp.float32))
    rst = rst * ndst_ref[...] + b_ref[...]
    mean = jnp.mean(rst, axis=0, keepdims=True)
    var = jnp.mean(rst * rst, axis=0, keepdims=True) - mean * mean
    o_ref[...] = ((rst - mean) * lax.rsqrt(var + EPS) * g_ref[...]
                  + be_ref[...] + x_ref[...])


_final_call = pl.pallas_call(
    _final_body,
    out_shape=jax.ShapeDtypeStruct((N, D), jnp.float32),
)


def kernel(x, edge_index, W, b, gamma, beta):
    src2 = edge_index[0].reshape(E // CHA, CHA)
    dst2 = edge_index[1].reshape(E // CHA, CHA)
    z1 = jnp.zeros((N,), jnp.float32)
    zh = jnp.zeros((N, DH), jnp.float32)

    deg = _deg_kernel(src2, dst2, z1)
    dsrc = deg[:N].reshape(N, 1)
    ddst = deg[N:].reshape(N, 1)
    flo, fhi, ndst = _feat_call(x, dsrc, ddst)
    part = _agg_kernel(flo, fhi, src2, dst2, zh).reshape(NC, N, DH)
    w2 = W.reshape(2, DH, D)
    return _final_call(part, w2, b.reshape(1, D), x, ndst,
                       gamma.reshape(1, D), beta.reshape(1, D))
